# baseline (speedup 1.0000x reference)
import functools

import jax
import jax.numpy as jnp
from jax.experimental import pallas as pl

E = 64
K = 2
CAP = 32.0


def _shift_rows(p, s, cols):
    if s > 0:
        return jnp.concatenate(
            [p[s:], jnp.zeros((s, cols), jnp.float32)], axis=0)
    if s < 0:
        return jnp.concatenate(
            [jnp.zeros((-s, cols), jnp.float32), p[:s]], axis=0)
    return p


def _tap_masks(Bb, H, W):
    hh = jax.lax.broadcasted_iota(jnp.int32, (Bb, H, W, 1), 1)
    ww = jax.lax.broadcasted_iota(jnp.int32, (Bb, H, W, 1), 2)
    N = Bb * H * W
    masks = {}
    for dy in (-1, 0, 1):
        for dx in (-1, 0, 1):
            masks[(dy, dx)] = ((hh + dy >= 0) & (hh + dy < H)
                               & (ww + dx >= 0)
                               & (ww + dx < W)).reshape(N, 1)
    return masks


def _conv3x3(x2, w, masks, H, W, Cin, Cout):
    N = x2.shape[0]
    acc = jnp.zeros((N, Cout), jnp.float32)
    for tap in range(9):
        dy, dx = tap // 3 - 1, tap % 3 - 1
        p = jnp.dot(x2, w[tap], preferred_element_type=jnp.float32)
        sp = _shift_rows(p, dy * W + dx, Cout)
        acc = acc + masks[(dy, dx)].astype(jnp.float32) * sp
    return acc


def _conv3x3_im2col(x2, w27, masks, H, W, Cin, Cout):
    N = x2.shape[0]
    pieces = []
    for tap in range(9):
        dy, dx = tap // 3 - 1, tap % 3 - 1
        sp = _shift_rows(x2, dy * W + dx, Cin)
        pieces.append(masks[(dy, dx)].astype(jnp.float32) * sp)
    patch = jnp.concatenate(pieces, axis=1)
    return jnp.dot(patch, w27, preferred_element_type=jnp.float32)


def _pool2(y, Bb, H, W, C):
    y = y.reshape(Bb, H // 2, 2, W, C).max(axis=2)
    return y.reshape(Bb, H // 2, W // 2, 2, C).max(axis=3).reshape(-1, C)


def _stage_kernel(x_ref, wa_ref, sa_ref, ta_ref, wb_ref, sb_ref, tb_ref,
                  o_ref, *, H, W, Ca, Cb, Cc, im2col, pool, gap):
    Bb = x_ref.shape[0]
    N = Bb * H * W
    x2 = x_ref[...].reshape(N, Ca)
    masks = _tap_masks(Bb, H, W)
    if im2col:
        acc = _conv3x3_im2col(x2, wa_ref[...], masks, H, W, Ca, Cb)
    else:
        acc = _conv3x3(x2, wa_ref[...], masks, H, W, Ca, Cb)
    y1 = jnp.maximum(acc * sa_ref[...] + ta_ref[...], 0.0)
    acc = _conv3x3(y1, wb_ref[...], masks, H, W, Cb, Cc)
    y2 = jnp.maximum(acc * sb_ref[...] + tb_ref[...], 0.0)
    if pool:
        o_ref[...] = _pool2(y2, Bb, H, W, Cc).reshape(
            Bb, H // 2, W // 2, Cc)
    elif gap:
        o_ref[...] = jnp.mean(y2.reshape(Bb, H * W, Cc), axis=1)
    else:
        o_ref[...] = y2.reshape(Bb, H, W, Cc)


def _fold_bn(blk):
    Cout = blk['w'].shape[0]
    w = blk['w'].transpose(2, 3, 1, 0).reshape(9, blk['w'].shape[1], Cout)
    s = blk['g'] / jnp.sqrt(blk['rv'] + 1e-5)
    t = (blk['b'] - blk['rm']) * s + blk['be']
    return w, s.reshape(1, Cout), t.reshape(1, Cout)


def _stage(x, blka, blkb, *, im2col, pool, gap, bb):
    B, H, W, Ca = x.shape
    wa, sa, ta = _fold_bn(blka)
    wb, sb, tb = _fold_bn(blkb)
    Cb, Cc = wa.shape[2], wb.shape[2]
    if im2col:
        wa = wa.reshape(9 * Ca, Cb)
        wa_spec = pl.BlockSpec((9 * Ca, Cb), lambda i: (0, 0))
    else:
        wa_spec = pl.BlockSpec((9, Ca, Cb), lambda i: (0, 0, 0))
    if gap:
        oshape, oblock = (B, Cc), (bb, Cc)
        omap = lambda i: (i, 0)
    elif pool:
        oshape, oblock = (B, H // 2, W // 2, Cc), (bb, H // 2, W // 2, Cc)
        omap = lambda i: (i, 0, 0, 0)
    else:
        oshape, oblock = (B, H, W, Cc), (bb, H, W, Cc)
        omap = lambda i: (i, 0, 0, 0)
    return pl.pallas_call(
        functools.partial(_stage_kernel, H=H, W=W, Ca=Ca, Cb=Cb, Cc=Cc,
                          im2col=im2col, pool=pool, gap=gap),
        grid=(B // bb,),
        in_specs=[
            pl.BlockSpec((bb, H, W, Ca), lambda i: (i, 0, 0, 0)),
            wa_spec,
            pl.BlockSpec((1, Cb), lambda i: (0, 0)),
            pl.BlockSpec((1, Cb), lambda i: (0, 0)),
            pl.BlockSpec((9, Cb, Cc), lambda i: (0, 0, 0)),
            pl.BlockSpec((1, Cc), lambda i: (0, 0)),
            pl.BlockSpec((1, Cc), lambda i: (0, 0)),
        ],
        out_specs=pl.BlockSpec(oblock, omap),
        out_shape=jax.ShapeDtypeStruct(oshape, jnp.float32),
    )(x, wa, sa, ta, wb, sb, tb)


def _trunk(x, tp):
    h = x.transpose(0, 2, 3, 1)
    h = _stage(h, tp[0], tp[1], im2col=False, pool=True, gap=False, bb=16)
    h = _stage(h, tp[2], tp[3], im2col=False, pool=True, gap=False, bb=32)
    h = _stage(h, tp[4], tp[5], im2col=False, pool=False, gap=True, bb=64)
    return h


def _ln(x, g, b):
    m = jnp.mean(x, axis=-1, keepdims=True)
    v = jnp.mean((x - m) ** 2, axis=-1, keepdims=True)
    return (x - m) / jnp.sqrt(v + 1e-5) * g + b


def _moe_kernel(f_ref, gw1_ref, gb1_ref, gg1_ref, gbe1_ref, gw2_ref, gb2_ref,
                gw3_ref, gb3_ref, ew1_ref, eb1_ref, eg1_ref, ebe1_ref,
                ew2_ref, eb2_ref, eg2_ref, ebe2_ref, ew3_ref, eb3_ref,
                cw_ref, cb_ref, u_ref, bal_ref, log_ref):
    f = f_ref[...]
    dot = lambda a, b: jnp.dot(a, b, preferred_element_type=jnp.float32)
    h = dot(f, gw1_ref[0]) + gb1_ref[0]
    h = jnp.maximum(_ln(h, gg1_ref[0], gbe1_ref[0]), 0.0)
    h = jnp.maximum(dot(h, gw2_ref[0]) + gb2_ref[0], 0.0)
    scores = dot(h, gw3_ref[0]) + gb3_ref[0]
    e1 = jnp.maximum(_ln(dot(f, ew1_ref[0]) + eb1_ref[0],
                         eg1_ref[0], ebe1_ref[0]), 0.0)
    e2 = jnp.maximum(_ln(dot(e1, ew2_ref[0]) + eb2_ref[0],
                         eg2_ref[0], ebe2_ref[0]), 0.0)
    emb = dot(e2, ew3_ref[0]) + eb3_ref[0]
    logits = dot(emb, cw_ref[0].T) + cb_ref[0]
    mx = jnp.max(logits, axis=-1, keepdims=True)
    ex = jnp.exp(logits - mx)
    z = jnp.sum(ex, axis=-1, keepdims=True)
    probs = ex / z
    ent = -jnp.sum(probs * jnp.log(probs + 1e-12), axis=-1, keepdims=True)
    usage = u_ref[0, 0, 0]
    boost = jnp.where(usage < 0.05, (0.05 - usage) * 10.0, 0.0)
    bal = 0.7 * scores + 0.3 * (-ent) + boost - 2.0 * usage
    bal_ref[...] = bal.reshape(1, 1, -1)
    log_ref[...] = logits[None]


def _moe(feats, g, ex, cls_w, cls_b, usage):
    B = feats.shape[0]
    v = lambda a: a.reshape(E, 1, -1)
    spec = lambda *blk: pl.BlockSpec(blk, lambda e: (e,) + (0,) * (len(blk) - 1))
    full = lambda *blk: pl.BlockSpec(blk, lambda e: (0,) * len(blk))
    bal_t, log_e = pl.pallas_call(
        _moe_kernel,
        grid=(E,),
        in_specs=[
            full(B, 256),
            spec(1, 256, 64), spec(1, 1, 64), spec(1, 1, 64), spec(1, 1, 64),
            spec(1, 64, 32), spec(1, 1, 32),
            spec(1, 32, 1), spec(1, 1, 1),
            spec(1, 256, 256), spec(1, 1, 256), spec(1, 1, 256),
            spec(1, 1, 256),
            spec(1, 256, 128), spec(1, 1, 128), spec(1, 1, 128),
            spec(1, 1, 128),
            spec(1, 128, 128), spec(1, 1, 128),
            spec(1, 10, 128), spec(1, 1, 10),
            spec(1, 1, 1),
        ],
        out_specs=[
            pl.BlockSpec((1, 1, B), lambda e: (e, 0, 0)),
            pl.BlockSpec((1, B, 10), lambda e: (e, 0, 0)),
        ],
        out_shape=[
            jax.ShapeDtypeStruct((E, 1, B), jnp.float32),
            jax.ShapeDtypeStruct((E, B, 10), jnp.float32),
        ],
    )(feats, g['W1'], v(g['b1']), v(g['g1']), v(g['be1']), g['W2'],
      v(g['b2']), g['W3'], v(g['b3']), ex['W1'], v(ex['b1']), v(ex['g1']),
      v(ex['be1']), ex['W2'], v(ex['b2']), v(ex['g2']), v(ex['be2']),
      ex['W3'], v(ex['b3']), cls_w, v(cls_b), usage.reshape(E, 1, 1))
    return bal_t.reshape(E, B), log_e


def _route_kernel(bal_ref, log_ref, d_ref, fin_ref):
    B = bal_ref.shape[0]
    iota = jax.lax.broadcasted_iota(jnp.int32, (1, E), 1)

    def body(i, loads):
        row = bal_ref[pl.ds(i, 1), :]
        m1 = jnp.max(row)
        i1 = jnp.min(jnp.where(row == m1, iota, E))
        masked = jnp.where(iota == i1, -jnp.inf, row)
        m2 = jnp.max(masked)
        i2 = jnp.min(jnp.where(masked == m2, iota, E))
        l1 = jnp.sum(jnp.where(iota == i1, loads, 0.0))
        l2 = jnp.sum(jnp.where(iota == i2, loads, 0.0))
        chosen = jnp.where(
            l1 < CAP, i1,
            jnp.where(l2 < CAP, i2, jnp.where(l1 <= l2, i1, i2)))
        oh = (iota == chosen).astype(jnp.float32)
        d_ref[pl.ds(i, 1), :] = oh
        return loads + oh

    jax.lax.fori_loop(0, B, body, jnp.zeros((1, E), jnp.float32))
    d = d_ref[...]
    fin_ref[...] = jnp.sum(d[:, :, None] * log_ref[...], axis=1)


def _route(balanced, logits_bec):
    B = balanced.shape[0]
    return pl.pallas_call(
        _route_kernel,
        out_shape=[
            jax.ShapeDtypeStruct((B, E), jnp.float32),
            jax.ShapeDtypeStruct((B, 10), jnp.float32),
        ],
    )(balanced, logits_bec)


def kernel(x, params):
    feats = _trunk(x, params['trunk'])
    g = params['gates']
    ex = params['experts']
    bal_t, log_e = _moe(feats, g, ex, params['cls_w'], params['cls_b'],
                        params['usage'])
    balanced = bal_t.T
    logits_bec = log_e.transpose(1, 0, 2)
    d, final = _route(balanced, logits_bec)
    return final, balanced, d > 0.5


# speedup vs baseline: 1.0129x; 1.0129x over previous
import functools

import jax
import jax.numpy as jnp
from jax.experimental import pallas as pl

E = 64
K = 2
CAP = 32.0


def _shift_rows(p, s, cols):
    if s > 0:
        return jnp.concatenate(
            [p[s:], jnp.zeros((s, cols), jnp.float32)], axis=0)
    if s < 0:
        return jnp.concatenate(
            [jnp.zeros((-s, cols), jnp.float32), p[:s]], axis=0)
    return p


def _tap_masks(Bb, H, W):
    hh = jax.lax.broadcasted_iota(jnp.int32, (Bb, H, W, 1), 1)
    ww = jax.lax.broadcasted_iota(jnp.int32, (Bb, H, W, 1), 2)
    N = Bb * H * W
    masks = {}
    for dy in (-1, 0, 1):
        for dx in (-1, 0, 1):
            masks[(dy, dx)] = ((hh + dy >= 0) & (hh + dy < H)
                               & (ww + dx >= 0)
                               & (ww + dx < W)).reshape(N, 1)
    return masks


def _conv3x3(x2, w, masks, H, W, Cin, Cout):
    N = x2.shape[0]
    acc = jnp.zeros((N, Cout), jnp.float32)
    for tap in range(9):
        dy, dx = tap // 3 - 1, tap % 3 - 1
        p = jnp.dot(x2, w[tap], preferred_element_type=jnp.float32)
        sp = _shift_rows(p, dy * W + dx, Cout)
        acc = acc + masks[(dy, dx)].astype(jnp.float32) * sp
    return acc


def _pool2(y, Bb, H, W, C):
    y = y.reshape(Bb, H // 2, 2, W, C).max(axis=2)
    return y.reshape(Bb, H // 2, W // 2, 2, C).max(axis=3).reshape(-1, C)


def _conv_kernel(x_ref, w_ref, s_ref, t_ref, o_ref, *, H, W, Cin, Cout,
                 pool, gap):
    Bb = x_ref.shape[0]
    N = Bb * H * W
    x2 = x_ref[...].reshape(N, Cin)
    masks = _tap_masks(Bb, H, W)
    acc = _conv3x3(x2, w_ref[...], masks, H, W, Cin, Cout)
    y = jnp.maximum(acc * s_ref[...] + t_ref[...], 0.0)
    if pool:
        o_ref[...] = _pool2(y, Bb, H, W, Cout).reshape(
            Bb, H // 2, W // 2, Cout)
    elif gap:
        o_ref[...] = jnp.mean(y.reshape(Bb, H * W, Cout), axis=1)
    else:
        o_ref[...] = y.reshape(Bb, H, W, Cout)


def _conv_layer(x, blk, *, pool, gap, bb):
    B, H, W, Cin = x.shape
    w, s, t = _fold_bn(blk)
    Cout = w.shape[2]
    if gap:
        oshape, oblock = (B, Cout), (bb, Cout)
        omap = lambda i: (i, 0)
    elif pool:
        oshape, oblock = (B, H // 2, W // 2, Cout), (bb, H // 2, W // 2, Cout)
        omap = lambda i: (i, 0, 0, 0)
    else:
        oshape, oblock = (B, H, W, Cout), (bb, H, W, Cout)
        omap = lambda i: (i, 0, 0, 0)
    return pl.pallas_call(
        functools.partial(_conv_kernel, H=H, W=W, Cin=Cin, Cout=Cout,
                          pool=pool, gap=gap),
        grid=(B // bb,),
        in_specs=[
            pl.BlockSpec((bb, H, W, Cin), lambda i: (i, 0, 0, 0)),
            pl.BlockSpec((9, Cin, Cout), lambda i: (0, 0, 0)),
            pl.BlockSpec((1, Cout), lambda i: (0, 0)),
            pl.BlockSpec((1, Cout), lambda i: (0, 0)),
        ],
        out_specs=pl.BlockSpec(oblock, omap),
        out_shape=jax.ShapeDtypeStruct(oshape, jnp.float32),
    )(x, w, s, t)


def _fold_bn(blk):
    Cout = blk['w'].shape[0]
    w = blk['w'].transpose(2, 3, 1, 0).reshape(9, blk['w'].shape[1], Cout)
    s = blk['g'] / jnp.sqrt(blk['rv'] + 1e-5)
    t = (blk['b'] - blk['rm']) * s + blk['be']
    return w, s.reshape(1, Cout), t.reshape(1, Cout)


def _trunk(x, tp):
    h = x.transpose(0, 2, 3, 1)
    h = _conv_layer(h, tp[0], pool=False, gap=False, bb=16)
    h = _conv_layer(h, tp[1], pool=True, gap=False, bb=16)
    h = _conv_layer(h, tp[2], pool=False, gap=False, bb=32)
    h = _conv_layer(h, tp[3], pool=True, gap=False, bb=32)
    h = _conv_layer(h, tp[4], pool=False, gap=False, bb=64)
    h = _conv_layer(h, tp[5], pool=False, gap=True, bb=64)
    return h


def _ln(x, g, b):
    m = jnp.mean(x, axis=-1, keepdims=True)
    v = jnp.mean((x - m) ** 2, axis=-1, keepdims=True)
    return (x - m) / jnp.sqrt(v + 1e-5) * g + b


def _moe_kernel(f_ref, gw1_ref, gb1_ref, gg1_ref, gbe1_ref, gw2_ref, gb2_ref,
                gw3_ref, gb3_ref, ew1_ref, eb1_ref, eg1_ref, ebe1_ref,
                ew2_ref, eb2_ref, eg2_ref, ebe2_ref, ew3_ref, eb3_ref,
                cw_ref, cb_ref, u_ref, bal_ref, log_ref):
    f = f_ref[...]
    dot = lambda a, b: jnp.dot(a, b, preferred_element_type=jnp.float32)
    h = dot(f, gw1_ref[0]) + gb1_ref[0]
    h = jnp.maximum(_ln(h, gg1_ref[0], gbe1_ref[0]), 0.0)
    h = jnp.maximum(dot(h, gw2_ref[0]) + gb2_ref[0], 0.0)
    scores = dot(h, gw3_ref[0]) + gb3_ref[0]
    e1 = jnp.maximum(_ln(dot(f, ew1_ref[0]) + eb1_ref[0],
                         eg1_ref[0], ebe1_ref[0]), 0.0)
    e2 = jnp.maximum(_ln(dot(e1, ew2_ref[0]) + eb2_ref[0],
                         eg2_ref[0], ebe2_ref[0]), 0.0)
    emb = dot(e2, ew3_ref[0]) + eb3_ref[0]
    logits = dot(emb, cw_ref[0].T) + cb_ref[0]
    mx = jnp.max(logits, axis=-1, keepdims=True)
    ex = jnp.exp(logits - mx)
    z = jnp.sum(ex, axis=-1, keepdims=True)
    probs = ex / z
    ent = -jnp.sum(probs * jnp.log(probs + 1e-12), axis=-1, keepdims=True)
    usage = u_ref[0, 0, 0]
    boost = jnp.where(usage < 0.05, (0.05 - usage) * 10.0, 0.0)
    bal = 0.7 * scores + 0.3 * (-ent) + boost - 2.0 * usage
    bal_ref[...] = bal.reshape(1, 1, -1)
    log_ref[...] = logits[None]


def _moe(feats, g, ex, cls_w, cls_b, usage):
    B = feats.shape[0]
    v = lambda a: a.reshape(E, 1, -1)
    spec = lambda *blk: pl.BlockSpec(blk, lambda e: (e,) + (0,) * (len(blk) - 1))
    full = lambda *blk: pl.BlockSpec(blk, lambda e: (0,) * len(blk))
    bal_t, log_e = pl.pallas_call(
        _moe_kernel,
        grid=(E,),
        in_specs=[
            full(B, 256),
            spec(1, 256, 64), spec(1, 1, 64), spec(1, 1, 64), spec(1, 1, 64),
            spec(1, 64, 32), spec(1, 1, 32),
            spec(1, 32, 1), spec(1, 1, 1),
            spec(1, 256, 256), spec(1, 1, 256), spec(1, 1, 256),
            spec(1, 1, 256),
            spec(1, 256, 128), spec(1, 1, 128), spec(1, 1, 128),
            spec(1, 1, 128),
            spec(1, 128, 128), spec(1, 1, 128),
            spec(1, 10, 128), spec(1, 1, 10),
            spec(1, 1, 1),
        ],
        out_specs=[
            pl.BlockSpec((1, 1, B), lambda e: (e, 0, 0)),
            pl.BlockSpec((1, B, 10), lambda e: (e, 0, 0)),
        ],
        out_shape=[
            jax.ShapeDtypeStruct((E, 1, B), jnp.float32),
            jax.ShapeDtypeStruct((E, B, 10), jnp.float32),
        ],
    )(feats, g['W1'], v(g['b1']), v(g['g1']), v(g['be1']), g['W2'],
      v(g['b2']), g['W3'], v(g['b3']), ex['W1'], v(ex['b1']), v(ex['g1']),
      v(ex['be1']), ex['W2'], v(ex['b2']), v(ex['g2']), v(ex['be2']),
      ex['W3'], v(ex['b3']), cls_w, v(cls_b), usage.reshape(E, 1, 1))
    return bal_t.reshape(E, B), log_e


def _route_kernel(bal_ref, log_ref, d_ref, fin_ref):
    B = bal_ref.shape[0]
    iota = jax.lax.broadcasted_iota(jnp.int32, (1, E), 1)

    def body(i, loads):
        row = bal_ref[pl.ds(i, 1), :]
        m1 = jnp.max(row)
        i1 = jnp.min(jnp.where(row == m1, iota, E))
        masked = jnp.where(iota == i1, -jnp.inf, row)
        m2 = jnp.max(masked)
        i2 = jnp.min(jnp.where(masked == m2, iota, E))
        l1 = jnp.sum(jnp.where(iota == i1, loads, 0.0))
        l2 = jnp.sum(jnp.where(iota == i2, loads, 0.0))
        chosen = jnp.where(
            l1 < CAP, i1,
            jnp.where(l2 < CAP, i2, jnp.where(l1 <= l2, i1, i2)))
        oh = (iota == chosen).astype(jnp.float32)
        d_ref[pl.ds(i, 1), :] = oh
        return loads + oh

    jax.lax.fori_loop(0, B, body, jnp.zeros((1, E), jnp.float32))
    d = d_ref[...]
    fin_ref[...] = jnp.sum(d[:, :, None] * log_ref[...], axis=1)


def _route(balanced, logits_bec):
    B = balanced.shape[0]
    return pl.pallas_call(
        _route_kernel,
        out_shape=[
            jax.ShapeDtypeStruct((B, E), jnp.float32),
            jax.ShapeDtypeStruct((B, 10), jnp.float32),
        ],
    )(balanced, logits_bec)


def kernel(x, params):
    feats = _trunk(x, params['trunk'])
    g = params['gates']
    ex = params['experts']
    bal_t, log_e = _moe(feats, g, ex, params['cls_w'], params['cls_b'],
                        params['usage'])
    balanced = bal_t.T
    logits_bec = log_e.transpose(1, 0, 2)
    d, final = _route(balanced, logits_bec)
    return final, balanced, d > 0.5
